# baseline (speedup 1.0000x reference)
import functools

import jax
import jax.numpy as jnp
from jax import lax
from jax.experimental import pallas as pl
from jax.experimental.pallas import tpu as pltpu
from jax.experimental.pallas import tpu_sc as plsc

_EPS = 1e-12

_D = 768
_BT = 16 * 1024
_S = 1024

_NC = 2
_NS = 16
_NW = _NC * _NS
_CHUNK = 64


def _sc_gather(table, idx, nrows):
    bpw = nrows // _NW
    nchunk = bpw // _CHUNK
    mesh = plsc.VectorSubcoreMesh(core_axis_name="c", subcore_axis_name="s")

    @functools.partial(
        pl.kernel,
        mesh=mesh,
        out_type=jax.ShapeDtypeStruct((nrows, _D), jnp.float32),
        scratch_types=[
            pltpu.VMEM((bpw,), jnp.int32),
            pltpu.VMEM((2, _CHUNK, _D), jnp.float32),
            pltpu.SemaphoreType.DMA,
            pltpu.SemaphoreType.DMA,
            pltpu.SemaphoreType.DMA,
            pltpu.SemaphoreType.DMA,
        ],
    )
    def gk(table_hbm, idx_hbm, out_hbm, idx_v, rows_v, gs0, gs1, os0, os1):
        gs = (gs0, gs1)
        osm = (os0, os1)
        wid = lax.axis_index("s") * _NC + lax.axis_index("c")
        base = wid * bpw
        pltpu.sync_copy(idx_hbm.at[pl.ds(base, bpw)], idx_v)

        def start_gather(j):
            b = j % 2
            return pltpu.async_copy(
                table_hbm.at[idx_v.at[pl.ds(j * _CHUNK, _CHUNK)]],
                rows_v.at[b], gs[b])

        g = [start_gather(0), start_gather(1)]
        for j in range(nchunk):
            b = j % 2
            g[b].wait()
            oc = pltpu.async_copy(
                rows_v.at[b],
                out_hbm.at[pl.ds(base + j * _CHUNK, _CHUNK)], osm[b])
            if j + 2 < nchunk:
                oc.wait()
                g[b] = start_gather(j + 2)
            else:
                oc.wait()

    return gk(table, idx)


def _finish_body_acc(acc_ref, g_ref, pos_ref, bits_ref, tt_ref, gamma_ref,
                     beta_ref, out_ref):
    del acc_ref
    _finish_body(g_ref, pos_ref, bits_ref, tt_ref, gamma_ref, beta_ref,
                 out_ref)


def _finish_body(g_ref, pos_ref, bits_ref, tt_ref, gamma_ref,
                 beta_ref, out_ref):
    tts = tt_ref[...]
    base = jnp.sum(tts[:, 0, :], axis=0)
    delta = tts[:, 1, :] - tts[:, 0, :]
    ttsum = jnp.dot(bits_ref[...], delta, preferred_element_type=jnp.float32,
                    precision=lax.Precision.HIGHEST)
    x = g_ref[...] + pos_ref[...] + base[None, :] + ttsum
    mean = jnp.mean(x, axis=-1, keepdims=True)
    msq = jnp.mean(x * x, axis=-1, keepdims=True)
    var = msq - mean * mean
    scale = lax.rsqrt(var + _EPS) * gamma_ref[...]
    out_ref[...] = x * scale - mean * scale + beta_ref[...]


def _tc_finish_chunk(acc, gathered_c, pos_emb, bits_c, tt_pairs, gamma, beta,
                     nb_chunk, row_off):
    rows = 1024
    grid = (nb_chunk,)
    boff = row_off // rows
    specs = [
        pl.BlockSpec((rows, _D), lambda b: (b, 0)),
        pl.BlockSpec((rows, _D), lambda b: (0, 0)),
        pl.BlockSpec((rows, 7), lambda b: (b, 0)),
        pl.BlockSpec((7, 2, _D), lambda b: (0, 0, 0)),
        pl.BlockSpec((1, _D), lambda b: (0, 0)),
        pl.BlockSpec((1, _D), lambda b: (0, 0)),
    ]
    out_spec = pl.BlockSpec((rows, _D), lambda b: (boff + b, 0))
    out_shape = jax.ShapeDtypeStruct((_BT, _D), jnp.float32)
    if acc is None:
        return pl.pallas_call(
            _finish_body, grid=grid, in_specs=specs, out_specs=out_spec,
            out_shape=out_shape,
        )(gathered_c, pos_emb, bits_c, tt_pairs, gamma, beta)
    return pl.pallas_call(
        _finish_body_acc, grid=grid,
        in_specs=[pl.BlockSpec(memory_space=pl.ANY)] + specs,
        out_specs=out_spec, out_shape=out_shape,
        input_output_aliases={0: 0},
    )(acc, gathered_c, pos_emb, bits_c, tt_pairs, gamma, beta)


def kernel(input_ids, token_type_ids, word_emb, pos_emb,
           tt_emb_0, tt_emb_1, tt_emb_2, tt_emb_3, tt_emb_4, tt_emb_5,
           tt_emb_6, ln_gamma, ln_beta):
    b, s = input_ids.shape
    ids = input_ids.reshape(-1).astype(jnp.int32)
    bits = token_type_ids.reshape(b * s, 7).astype(jnp.float32)
    tt_pairs = jnp.stack([
        tt_emb_0[0:2], tt_emb_1[0:2], tt_emb_2[0:2], tt_emb_3[0:2],
        tt_emb_4[0:2], tt_emb_5[0:2], tt_emb_6[0:2]])
    gamma = ln_gamma.reshape(1, _D)
    beta = ln_beta.reshape(1, _D)

    nchunks = 2
    cbt = _BT // nchunks
    gathered = [_sc_gather(word_emb, ids[c * cbt:(c + 1) * cbt], cbt)
                for c in range(nchunks)]
    acc = None
    for c in range(nchunks):
        acc = _tc_finish_chunk(acc, gathered[c], pos_emb,
                               bits[c * cbt:(c + 1) * cbt], tt_pairs,
                               gamma, beta, cbt // _S, c * cbt)
    return acc.reshape(b, s, _D)


# speedup vs baseline: 1.0348x; 1.0348x over previous
import functools

import jax
import jax.numpy as jnp
from jax import lax
from jax.experimental import pallas as pl
from jax.experimental.pallas import tpu as pltpu
from jax.experimental.pallas import tpu_sc as plsc

_EPS = 1e-12

_D = 768
_BT = 16 * 1024
_S = 1024

_NC = 2
_NS = 16
_NW = _NC * _NS
_CHUNK = 64


def _sc_gather(table, idx, nrows):
    bpw = nrows // _NW
    nchunk = bpw // _CHUNK
    mesh = plsc.VectorSubcoreMesh(core_axis_name="c", subcore_axis_name="s")

    @functools.partial(
        pl.kernel,
        mesh=mesh,
        out_type=jax.ShapeDtypeStruct((nrows, _D), jnp.float32),
        scratch_types=[
            pltpu.VMEM((bpw,), jnp.int32),
            pltpu.VMEM((2, _CHUNK, _D), jnp.float32),
            pltpu.SemaphoreType.DMA,
            pltpu.SemaphoreType.DMA,
            pltpu.SemaphoreType.DMA,
            pltpu.SemaphoreType.DMA,
        ],
    )
    def gk(table_hbm, idx_hbm, out_hbm, idx_v, rows_v, gs0, gs1, os0, os1):
        gs = (gs0, gs1)
        osm = (os0, os1)
        wid = lax.axis_index("s") * _NC + lax.axis_index("c")
        base = wid * bpw
        pltpu.sync_copy(idx_hbm.at[pl.ds(base, bpw)], idx_v)

        def start_gather(j):
            b = j % 2
            return pltpu.async_copy(
                table_hbm.at[idx_v.at[pl.ds(j * _CHUNK, _CHUNK)]],
                rows_v.at[b], gs[b])

        g = [start_gather(0), start_gather(1)]
        for j in range(nchunk):
            b = j % 2
            g[b].wait()
            oc = pltpu.async_copy(
                rows_v.at[b],
                out_hbm.at[pl.ds(base + j * _CHUNK, _CHUNK)], osm[b])
            if j + 2 < nchunk:
                oc.wait()
                g[b] = start_gather(j + 2)
            else:
                oc.wait()

    return gk(table, idx)


def _finish_body(g_ref, pos_ref, bits_ref, tt_ref, gamma_ref,
                 beta_ref, out_ref):
    tts = tt_ref[...]
    base = jnp.sum(tts[:, 0, :], axis=0)
    delta = tts[:, 1, :] - tts[:, 0, :]
    ttsum = jnp.dot(bits_ref[...], delta, preferred_element_type=jnp.float32,
                    precision=lax.Precision.HIGHEST)
    x = g_ref[...] + pos_ref[...] + base[None, :] + ttsum
    mean = jnp.mean(x, axis=-1, keepdims=True)
    msq = jnp.mean(x * x, axis=-1, keepdims=True)
    var = msq - mean * mean
    scale = lax.rsqrt(var + _EPS) * gamma_ref[...]
    out_ref[...] = x * scale - mean * scale + beta_ref[...]


def _tc_finish(gathered, pos_emb, bits, tt_pairs, gamma, beta, rows=1024):
    per_seq = _S // rows
    nb = _BT // _S
    grid = (per_seq, nb)
    return pl.pallas_call(
        _finish_body,
        grid=grid,
        in_specs=[
            pl.BlockSpec((rows, _D), lambda p, b: (b * per_seq + p, 0)),
            pl.BlockSpec((rows, _D), lambda p, b: (p, 0)),
            pl.BlockSpec((rows, 7), lambda p, b: (b * per_seq + p, 0)),
            pl.BlockSpec((7, 2, _D), lambda p, b: (0, 0, 0)),
            pl.BlockSpec((1, _D), lambda p, b: (0, 0)),
            pl.BlockSpec((1, _D), lambda p, b: (0, 0)),
        ],
        out_specs=pl.BlockSpec((rows, _D), lambda p, b: (b * per_seq + p, 0)),
        out_shape=jax.ShapeDtypeStruct((_BT, _D), jnp.float32),
    )(gathered, pos_emb, bits, tt_pairs, gamma, beta)


def kernel(input_ids, token_type_ids, word_emb, pos_emb,
           tt_emb_0, tt_emb_1, tt_emb_2, tt_emb_3, tt_emb_4, tt_emb_5,
           tt_emb_6, ln_gamma, ln_beta):
    b, s = input_ids.shape
    ids = input_ids.reshape(-1).astype(jnp.int32)
    bits = token_type_ids.reshape(b * s, 7).astype(jnp.float32)
    tt_pairs = jnp.stack([
        tt_emb_0[0:2], tt_emb_1[0:2], tt_emb_2[0:2], tt_emb_3[0:2],
        tt_emb_4[0:2], tt_emb_5[0:2], tt_emb_6[0:2]])
    gamma = ln_gamma.reshape(1, _D)
    beta = ln_beta.reshape(1, _D)

    gathered = _sc_gather(word_emb, ids, _BT)
    out = _tc_finish(gathered, pos_emb, bits, tt_pairs, gamma, beta)
    return out.reshape(b, s, _D)
